# baseline (speedup 1.0000x reference)
import functools

import jax
import jax.numpy as jnp
from jax import lax
from jax.experimental import pallas as pl
from jax.experimental.pallas import tpu as pltpu
from jax.experimental.pallas import tpu_sc as plsc

_B = 16384
_D = 64
_NW = 32
_BPW = _B // _NW
_CHUNK = 128
_NCHUNK = _BPW // _CHUNK


def _sc_body(uidx_hbm, iidx_hbm, user_hbm, item_hbm, pout_hbm,
             uidx_v, iidx_v, urows_v, irows_v, pout_v, sem):
    wid = lax.axis_index("s") * 2 + lax.axis_index("c")
    base = wid * _BPW

    pltpu.sync_copy(uidx_hbm.at[pl.ds(base, _BPW)], uidx_v)
    pltpu.sync_copy(iidx_hbm.at[pl.ds(base, _BPW)], iidx_v)

    copies = []
    for j in range(_NCHUNK):
        idx = uidx_v.at[pl.ds(j * _CHUNK, _CHUNK)]
        dst = urows_v.at[pl.ds(j * _CHUNK, _CHUNK)]
        copies.append(pltpu.make_async_copy(user_hbm.at[idx], dst, sem))
    for j in range(_NCHUNK):
        idx = iidx_v.at[pl.ds(j * _CHUNK, _CHUNK)]
        dst = irows_v.at[pl.ds(j * _CHUNK, _CHUNK)]
        copies.append(pltpu.make_async_copy(item_hbm.at[idx], dst, sem))
    for c in copies:
        c.start()
    for c in copies:
        c.wait()

    def body(g, carry):
        for rr in range(4):
            r = g * 4 + rr
            s = urows_v[r, pl.ds(0, 16)] * irows_v[r, pl.ds(0, 16)]
            for k in range(1, _D // 16):
                s = s + urows_v[r, pl.ds(k * 16, 16)] * irows_v[r, pl.ds(k * 16, 16)]
            pout_v[r, pl.ds(0, 16)] = s
        return carry

    lax.fori_loop(0, _BPW // 4, body, 0)
    pltpu.sync_copy(pout_v, pout_hbm.at[pl.ds(base, _BPW)])


def _tc_body(p_ref, o_ref):
    r = lax.broadcasted_iota(jnp.int32, (128, 8), 0)
    c = lax.broadcasted_iota(jnp.int32, (128, 8), 1)
    sel = (r // 16 == c).astype(jnp.float32)
    o_ref[...] = jnp.dot(p_ref[...], sel, preferred_element_type=jnp.float32)


def kernel(inputs, user_embedding, item_embedding):
    mesh = plsc.VectorSubcoreMesh(core_axis_name="c", subcore_axis_name="s")
    sc_run = functools.partial(
        pl.kernel,
        out_type=jax.ShapeDtypeStruct((_B, 16), jnp.float32),
        mesh=mesh,
        compiler_params=pltpu.CompilerParams(use_tc_tiling_on_sc=False),
        scratch_types=[
            pltpu.VMEM((_BPW,), jnp.int32),
            pltpu.VMEM((_BPW,), jnp.int32),
            pltpu.VMEM((_BPW, _D), jnp.float32),
            pltpu.VMEM((_BPW, _D), jnp.float32),
            pltpu.VMEM((_BPW, 16), jnp.float32),
            pltpu.SemaphoreType.DMA,
        ],
    )(_sc_body)
    uidx = inputs[:, 0].reshape(_B)
    iidx = inputs[:, 1].reshape(_B)
    partials = sc_run(uidx, iidx, user_embedding, item_embedding)
    out = pl.pallas_call(
        _tc_body,
        out_shape=jax.ShapeDtypeStruct((_B // 8, 8), jnp.float32),
    )(partials.reshape(_B // 8, 128))
    return out.reshape(_B)
